# baseline (speedup 1.0000x reference)
import functools

import jax
import jax.numpy as jnp
from jax import lax
from jax.experimental import pallas as pl
from jax.experimental.pallas import tpu as pltpu
from jax.experimental.pallas import tpu_sc as plsc

B = 64
N = 20000
NP = 20480
NCHUNK = 80
CS = 256
Q = 200
K = 256
ROWS_PER_STEP = 8
D_FEAT = 16
MODEL_DIM = 256
NUM_CLASSES = 10
INT_MIN = -(2 ** 31)


def _order_first(ka, ia, kb, ib):
    return (ka > kb) | ((ka == kb) & (ia < ib))


def _bitonic_stage(key, idx, j, dirmask):
    lane = lax.broadcasted_iota(jnp.int32, key.shape, key.ndim - 1)
    first = (lane & j) == 0
    pk = jnp.where(first, pltpu.roll(key, CS - j, key.ndim - 1),
                   pltpu.roll(key, j, key.ndim - 1))
    pi = jnp.where(first, pltpu.roll(idx, CS - j, key.ndim - 1),
                   pltpu.roll(idx, j, key.ndim - 1))
    mine_first = _order_first(key, idx, pk, pi)
    take_mine = mine_first == (first == dirmask)
    return jnp.where(take_mine, key, pk), jnp.where(take_mine, idx, pi)


def _sort_chunks(key, idx):
    lane = lax.broadcasted_iota(jnp.int32, key.shape, key.ndim - 1)
    k = 2
    while k <= CS:
        dirmask = (lane & k) == 0
        j = k // 2
        while j >= 1:
            key, idx = _bitonic_stage(key, idx, j, dirmask)
            j //= 2
        k *= 2
    return key, idx


def _flip_lanes(x):
    lane = lax.broadcasted_iota(jnp.int32, x.shape, x.ndim - 1)
    j = CS // 2
    while j >= 1:
        x = jnp.where((lane & j) == 0, pltpu.roll(x, CS - j, x.ndim - 1),
                      pltpu.roll(x, j, x.ndim - 1))
        j //= 2
    return x


def _merge_pair(ak, ai, bk, bi):
    bk = _flip_lanes(bk)
    bi = _flip_lanes(bi)
    a_wins = _order_first(ak, ai, bk, bi)
    mk = jnp.where(a_wins, ak, bk)
    mi = jnp.where(a_wins, ai, bi)
    true_dir = jnp.ones(mk.shape, dtype=bool)
    j = CS // 2
    while j >= 1:
        mk, mi = _bitonic_stage(mk, mi, j, true_dir)
        j //= 2
    return mk, mi


def _topk_body(scores_ref, valid_ref, out_scores_ref, out_flat_ref):
    s = scores_ref[...]
    v = valid_ref[...]
    y = lax.bitcast_convert_type(s, jnp.int32)
    z = y ^ jnp.right_shift(y, 31).astype(jnp.int32) & jnp.int32(0x7FFFFFFF)
    key = jnp.where(v > 0, z, jnp.int32(INT_MIN))
    idx = (lax.broadcasted_iota(jnp.int32, key.shape, 1) * CS
           + lax.broadcasted_iota(jnp.int32, key.shape, 2))
    key, idx = _sort_chunks(key, idx)
    nlists = NCHUNK
    while nlists > 1:
        h = nlists // 2
        ak, ai = key[:, :h], idx[:, :h]
        bk, bi = key[:, h:2 * h], idx[:, h:2 * h]
        mk, mi = _merge_pair(ak, ai, bk, bi)
        if nlists % 2:
            key = jnp.concatenate([mk, key[:, 2 * h:]], axis=1)
            idx = jnp.concatenate([mi, idx[:, 2 * h:]], axis=1)
            nlists = h + 1
        else:
            key, idx = mk, mi
            nlists = h
    kk = key[:, 0, :]
    ii = idx[:, 0, :]
    yk = kk ^ jnp.right_shift(kk, 31).astype(jnp.int32) & jnp.int32(0x7FFFFFFF)
    out_scores_ref[...] = lax.bitcast_convert_type(yk, jnp.float32)
    row = (pl.program_id(0) * ROWS_PER_STEP
           + lax.broadcasted_iota(jnp.int32, ii.shape, 0))
    out_flat_ref[...] = row * N + ii


def _topk_call(scores3, valid3):
    grid = B // ROWS_PER_STEP
    return pl.pallas_call(
        _topk_body,
        grid=(grid,),
        in_specs=[
            pl.BlockSpec((ROWS_PER_STEP, NCHUNK, CS), lambda i: (i, 0, 0)),
            pl.BlockSpec((ROWS_PER_STEP, NCHUNK, CS), lambda i: (i, 0, 0)),
        ],
        out_specs=[
            pl.BlockSpec((ROWS_PER_STEP, K), lambda i: (i, 0)),
            pl.BlockSpec((ROWS_PER_STEP, K), lambda i: (i, 0)),
        ],
        out_shape=[
            jax.ShapeDtypeStruct((B, K), jnp.float32),
            jax.ShapeDtypeStruct((B, K), jnp.int32),
        ],
    )(scores3, valid3)


_NW = 32
_BPW = (B * K) // _NW
_ICH = 128
_NCH = _BPW // _ICH


def _sc_gather_body(table_hbm, idx_hbm, out_hbm, idx_v, rows_v, sem):
    wid = lax.axis_index("s") * 2 + lax.axis_index("c")
    pltpu.sync_copy(idx_hbm.at[wid], idx_v)
    copies = [
        pltpu.async_copy(
            table_hbm.at[idx_v.at[ch]],
            rows_v.at[pl.ds(ch * _ICH, _ICH)],
            sem,
        )
        for ch in range(_NCH)
    ]
    for cp in copies:
        cp.wait()
    pltpu.sync_copy(rows_v, out_hbm.at[pl.ds(wid * _BPW, _BPW)])


def _sc_gather(table, idx3):
    mesh = plsc.VectorSubcoreMesh(core_axis_name="c", subcore_axis_name="s")
    return pl.kernel(
        _sc_gather_body,
        out_type=jax.ShapeDtypeStruct((B * K, D_FEAT), jnp.float32),
        mesh=mesh,
        compiler_params=pltpu.CompilerParams(use_tc_tiling_on_sc=False),
        scratch_types=[
            pltpu.VMEM((_NCH, _ICH), jnp.int32),
            pltpu.VMEM((_BPW, D_FEAT), jnp.float32),
            pltpu.SemaphoreType.DMA,
        ],
    )(table, idx3)


def _mlp_body(g_ref, w1t_ref, b1_ref, w2t_ref, b2_ref, q_ref, r_ref):
    g = g_ref[...].reshape(ROWS_PER_STEP * K, D_FEAT)
    boxes9 = g[:, 0:9]
    score1 = g[:, 9:10]
    label1 = g[:, 10:11]
    oh = jnp.where(
        label1.astype(jnp.int32)
        == lax.broadcasted_iota(jnp.int32, (ROWS_PER_STEP * K, NUM_CLASSES), 1),
        jnp.float32(1.0), jnp.float32(0.0))
    feat = jnp.concatenate([boxes9, score1, oh], axis=1)
    h = jnp.maximum(
        jnp.dot(feat, w1t_ref[...], preferred_element_type=jnp.float32)
        + b1_ref[...], 0.0)
    o = (jnp.dot(h, w2t_ref[...], preferred_element_type=jnp.float32)
         + b2_ref[...])
    q_ref[...] = o.reshape(ROWS_PER_STEP, K, MODEL_DIM)[:, :Q, :]
    r_ref[...] = boxes9.reshape(ROWS_PER_STEP, K, 9)[:, :Q, :3]


def _mlp_call(g3, w1t, b1r, w2t, b2r):
    grid = B // ROWS_PER_STEP
    return pl.pallas_call(
        _mlp_body,
        grid=(grid,),
        in_specs=[
            pl.BlockSpec((ROWS_PER_STEP, K, D_FEAT), lambda i: (i, 0, 0)),
            pl.BlockSpec((20, MODEL_DIM), lambda i: (0, 0)),
            pl.BlockSpec((1, MODEL_DIM), lambda i: (0, 0)),
            pl.BlockSpec((MODEL_DIM, MODEL_DIM), lambda i: (0, 0)),
            pl.BlockSpec((1, MODEL_DIM), lambda i: (0, 0)),
        ],
        out_specs=[
            pl.BlockSpec((ROWS_PER_STEP, Q, MODEL_DIM), lambda i: (i, 0, 0)),
            pl.BlockSpec((ROWS_PER_STEP, Q, 3), lambda i: (i, 0, 0)),
        ],
        out_shape=[
            jax.ShapeDtypeStruct((B, Q, MODEL_DIM), jnp.float32),
            jax.ShapeDtypeStruct((B, Q, 3), jnp.float32),
        ],
    )(g3, w1t, b1r, w2t, b2r)


def kernel(object_boxes, object_scores, object_labels, valid_mask, W1, b1, W2, b2):
    scores3 = jnp.pad(object_scores, ((0, 0), (0, NP - N))).reshape(B, NCHUNK, CS)
    valid3 = jnp.pad(valid_mask.astype(jnp.int32),
                     ((0, 0), (0, NP - N))).reshape(B, NCHUNK, CS)
    scores_k, flat_idx = _topk_call(scores3, valid3)

    table = jnp.concatenate(
        [object_boxes,
         object_scores[..., None],
         object_labels.astype(jnp.float32)[..., None],
         jnp.zeros((B, N, D_FEAT - 11), jnp.float32)],
        axis=-1).reshape(B * N, D_FEAT)
    g = _sc_gather(table, flat_idx.reshape(_NW, _NCH, _ICH))

    queries, refs = _mlp_call(
        g.reshape(B, K, D_FEAT), W1.T, b1.reshape(1, -1), W2.T, b2.reshape(1, -1))
    return (queries, refs, scores_k[:, :Q])


# speedup vs baseline: 1.0876x; 1.0876x over previous
import functools

import jax
import jax.numpy as jnp
from jax import lax
from jax.experimental import pallas as pl
from jax.experimental.pallas import tpu as pltpu
from jax.experimental.pallas import tpu_sc as plsc

B = 64
N = 20000
NP = 20480
NCHUNK = 80
CS = 256
Q = 200
K = 256
ROWS_PER_STEP = 8
D_FEAT = 16
MODEL_DIM = 256
NUM_CLASSES = 10
INT_MIN = -(2 ** 31)


def _order_first(ka, ia, kb, ib):
    return (ka > kb) | ((ka == kb) & (ia < ib))


def _stage(key, idx, j, dm):
    s = key.shape
    if j == 128:
        sub = lax.broadcasted_iota(jnp.int32, s, 1)
        first = (sub & 1) == 0
        nsub = s[1]
        pk = jnp.where(first, pltpu.roll(key, nsub - 1, 1), pltpu.roll(key, 1, 1))
        pi = jnp.where(first, pltpu.roll(idx, nsub - 1, 1), pltpu.roll(idx, 1, 1))
    else:
        lane = lax.broadcasted_iota(jnp.int32, s, 2)
        first = (lane & j) == 0
        pk = jnp.where(first, pltpu.roll(key, 128 - j, 2), pltpu.roll(key, j, 2))
        pi = jnp.where(first, pltpu.roll(idx, 128 - j, 2), pltpu.roll(idx, j, 2))
    take_mine = _order_first(key, idx, pk, pi) == (first == dm)
    return jnp.where(take_mine, key, pk), jnp.where(take_mine, idx, pi)


def _desc_mask(shape, nlists):
    sub = lax.broadcasted_iota(jnp.int32, shape, 1)
    return sub < 2 * ((nlists + 1) // 2)


def _sort_lists(key, idx):
    s = key.shape
    nlists = s[1] // 2
    cdesc = _desc_mask(s, nlists)
    sub = lax.broadcasted_iota(jnp.int32, s, 1)
    lane = lax.broadcasted_iota(jnp.int32, s, 2)
    k = 2
    while k <= 256:
        if k == 256:
            kbit = jnp.ones(s, dtype=bool)
        elif k == 128:
            kbit = (sub & 1) == 0
        else:
            kbit = (lane & k) == 0
        dm = kbit == cdesc
        j = k // 2
        while j >= 1:
            key, idx = _stage(key, idx, j, dm)
            j //= 2
        k *= 2
    return key, idx


def _pad_lists(key, idx, want_sub):
    extra = want_sub - key.shape[1]
    if extra <= 0:
        return key, idx
    shp = (key.shape[0], extra, key.shape[2])
    return (jnp.concatenate([key, jnp.full(shp, INT_MIN, jnp.int32)], axis=1),
            jnp.concatenate([idx, jnp.zeros(shp, jnp.int32)], axis=1))


def _merge_level(key, idx):
    s1 = key.shape[1]
    half = s1 // 2
    ak, ai = key[:, :half], idx[:, :half]
    bk, bi = key[:, half:], idx[:, half:]
    a_wins = _order_first(ak, ai, bk, bi)
    mk = jnp.where(a_wins, ak, bk)
    mi = jnp.where(a_wins, ai, bi)
    nlists = half // 2
    dm = _desc_mask(mk.shape, nlists)
    j = 128
    while j >= 1:
        mk, mi = _stage(mk, mi, j, dm)
        j //= 2
    return mk, mi


def _topk_body(scores_ref, valid_ref, out_scores_ref, out_flat_ref):
    sc = scores_ref[...]
    v = valid_ref[...]
    y = lax.bitcast_convert_type(sc, jnp.int32)
    z = y ^ jnp.right_shift(y, 31).astype(jnp.int32) & jnp.int32(0x7FFFFFFF)
    key = jnp.where(v > 0, z, jnp.int32(INT_MIN))
    idx = (lax.broadcasted_iota(jnp.int32, key.shape, 1) * 128
           + lax.broadcasted_iota(jnp.int32, key.shape, 2))
    key, idx = _sort_lists(key, idx)
    nlists = NCHUNK
    while nlists > 1:
        if nlists % 2:
            key, idx = _pad_lists(key, idx, (nlists + 1) * 2)
            nlists += 1
        key, idx = _merge_level(key, idx)
        nlists //= 2
    kk = key[:, 0:2, :].reshape(ROWS_PER_STEP, K)
    ii = idx[:, 0:2, :].reshape(ROWS_PER_STEP, K)
    yk = kk ^ jnp.right_shift(kk, 31).astype(jnp.int32) & jnp.int32(0x7FFFFFFF)
    out_scores_ref[...] = lax.bitcast_convert_type(yk, jnp.float32)
    row = (pl.program_id(0) * ROWS_PER_STEP
           + lax.broadcasted_iota(jnp.int32, ii.shape, 0))
    out_flat_ref[...] = row * N + ii


def _topk_call(scores3, valid3):
    grid = B // ROWS_PER_STEP
    return pl.pallas_call(
        _topk_body,
        grid=(grid,),
        in_specs=[
            pl.BlockSpec((ROWS_PER_STEP, 2 * NCHUNK, 128), lambda i: (i, 0, 0)),
            pl.BlockSpec((ROWS_PER_STEP, 2 * NCHUNK, 128), lambda i: (i, 0, 0)),
        ],
        out_specs=[
            pl.BlockSpec((ROWS_PER_STEP, K), lambda i: (i, 0)),
            pl.BlockSpec((ROWS_PER_STEP, K), lambda i: (i, 0)),
        ],
        out_shape=[
            jax.ShapeDtypeStruct((B, K), jnp.float32),
            jax.ShapeDtypeStruct((B, K), jnp.int32),
        ],
    )(scores3, valid3)


_NW = 32
_BPW = (B * K) // _NW
_ICH = 128
_NCH = _BPW // _ICH


def _sc_gather_body(table_hbm, idx_hbm, out_hbm, idx_v, rows_v, sem):
    wid = lax.axis_index("s") * 2 + lax.axis_index("c")
    pltpu.sync_copy(idx_hbm.at[wid], idx_v)
    copies = [
        pltpu.async_copy(
            table_hbm.at[idx_v.at[ch]],
            rows_v.at[pl.ds(ch * _ICH, _ICH)],
            sem,
        )
        for ch in range(_NCH)
    ]
    for cp in copies:
        cp.wait()
    pltpu.sync_copy(rows_v, out_hbm.at[pl.ds(wid * _BPW, _BPW)])


def _sc_gather(table, idx3):
    mesh = plsc.VectorSubcoreMesh(core_axis_name="c", subcore_axis_name="s")
    return pl.kernel(
        _sc_gather_body,
        out_type=jax.ShapeDtypeStruct((B * K, D_FEAT), jnp.float32),
        mesh=mesh,
        compiler_params=pltpu.CompilerParams(use_tc_tiling_on_sc=False),
        scratch_types=[
            pltpu.VMEM((_NCH, _ICH), jnp.int32),
            pltpu.VMEM((_BPW, D_FEAT), jnp.float32),
            pltpu.SemaphoreType.DMA,
        ],
    )(table, idx3)


def _mlp_body(g_ref, w1t_ref, b1_ref, w2t_ref, b2_ref, q_ref, r_ref):
    g = g_ref[...].reshape(ROWS_PER_STEP * K, D_FEAT)
    boxes9 = g[:, 0:9]
    score1 = g[:, 9:10]
    label1 = g[:, 10:11]
    oh = jnp.where(
        label1.astype(jnp.int32)
        == lax.broadcasted_iota(jnp.int32, (ROWS_PER_STEP * K, NUM_CLASSES), 1),
        jnp.float32(1.0), jnp.float32(0.0))
    feat = jnp.concatenate([boxes9, score1, oh], axis=1)
    h = jnp.maximum(
        jnp.dot(feat, w1t_ref[...], preferred_element_type=jnp.float32)
        + b1_ref[...], 0.0)
    o = (jnp.dot(h, w2t_ref[...], preferred_element_type=jnp.float32)
         + b2_ref[...])
    q_ref[...] = o.reshape(ROWS_PER_STEP, K, MODEL_DIM)[:, :Q, :]
    r_ref[...] = boxes9.reshape(ROWS_PER_STEP, K, 9)[:, :Q, :3]


def _mlp_call(g3, w1t, b1r, w2t, b2r):
    grid = B // ROWS_PER_STEP
    return pl.pallas_call(
        _mlp_body,
        grid=(grid,),
        in_specs=[
            pl.BlockSpec((ROWS_PER_STEP, K, D_FEAT), lambda i: (i, 0, 0)),
            pl.BlockSpec((20, MODEL_DIM), lambda i: (0, 0)),
            pl.BlockSpec((1, MODEL_DIM), lambda i: (0, 0)),
            pl.BlockSpec((MODEL_DIM, MODEL_DIM), lambda i: (0, 0)),
            pl.BlockSpec((1, MODEL_DIM), lambda i: (0, 0)),
        ],
        out_specs=[
            pl.BlockSpec((ROWS_PER_STEP, Q, MODEL_DIM), lambda i: (i, 0, 0)),
            pl.BlockSpec((ROWS_PER_STEP, Q, 3), lambda i: (i, 0, 0)),
        ],
        out_shape=[
            jax.ShapeDtypeStruct((B, Q, MODEL_DIM), jnp.float32),
            jax.ShapeDtypeStruct((B, Q, 3), jnp.float32),
        ],
    )(g3, w1t, b1r, w2t, b2r)


def kernel(object_boxes, object_scores, object_labels, valid_mask, W1, b1, W2, b2):
    scores3 = jnp.pad(object_scores, ((0, 0), (0, NP - N))).reshape(B, 2 * NCHUNK, 128)
    valid3 = jnp.pad(valid_mask.astype(jnp.int32),
                     ((0, 0), (0, NP - N))).reshape(B, 2 * NCHUNK, 128)
    scores_k, flat_idx = _topk_call(scores3, valid3)

    table = jnp.concatenate(
        [object_boxes,
         object_scores[..., None],
         object_labels.astype(jnp.float32)[..., None],
         jnp.zeros((B, N, D_FEAT - 11), jnp.float32)],
        axis=-1).reshape(B * N, D_FEAT)
    g = _sc_gather(table, flat_idx.reshape(_NW, _NCH, _ICH))

    queries, refs = _mlp_call(
        g.reshape(B, K, D_FEAT), W1.T, b1.reshape(1, -1), W2.T, b2.reshape(1, -1))
    return (queries, refs, scores_k[:, :Q])
